# baseline (speedup 1.0000x reference)
import functools

import jax
import jax.numpy as jnp
from jax import lax
from jax.experimental import pallas as pl
from jax.experimental.pallas import tpu as pltpu
from jax.experimental.pallas import tpu_sc as plsc

BATCH = 16384
DIM = 16
ROWS_PER_BLOCK = 8

_info = plsc.get_sparse_core_info()
_NC, _NS, _L = _info.num_cores, _info.num_subcores, _info.num_lanes
_NW = _NC * _NS
_B_PER_W = BATCH // _NW
_GROUPS = _B_PER_W // _L


def _make_gather():
    mesh = plsc.VectorSubcoreMesh(core_axis_name="c", subcore_axis_name="s")

    @functools.partial(
        pl.kernel,
        mesh=mesh,
        out_type=jax.ShapeDtypeStruct((BATCH * DIM,), jnp.float32),
        scratch_types=[
            pltpu.VMEM((_B_PER_W,), jnp.int32),
            pltpu.VMEM((_B_PER_W,), jnp.int32),
            pltpu.VMEM((_B_PER_W, 128), jnp.float32),
            pltpu.VMEM((_B_PER_W * DIM,), jnp.float32),
            pltpu.SemaphoreType.DMA,
        ],
        compiler_params=pltpu.CompilerParams(
            use_tc_tiling_on_sc=False, needs_layout_passes=False
        ),
    )
    def gather_kernel(idx_hbm, table_hbm, out_hbm, idx_v, blk_v, rows_v,
                      out_v, sem):
        wid = lax.axis_index("s") * _NC + lax.axis_index("c")
        base = wid * _B_PER_W
        pltpu.sync_copy(idx_hbm.at[pl.ds(base, _B_PER_W)], idx_v)

        def compute_blocks(g):
            v = idx_v[pl.ds(g * _L, _L)]
            blk_v[pl.ds(g * _L, _L)] = v >> 3
        pl.loop(0, _GROUPS)(compute_blocks)

        pltpu.async_copy(table_hbm.at[blk_v], rows_v, sem).wait()

        lanes = lax.iota(jnp.int32, _L)

        def extract(g):
            v = idx_v[pl.ds(g * _L, _L)]
            sub = (v & 7) * DIM
            rowsel = g * _L + lanes
            outbase = (g * _L + lanes) * DIM
            for d in range(DIM):
                vals = plsc.load_gather(rows_v, [rowsel, sub + d])
                plsc.store_scatter(out_v, [outbase + d], vals)
        pl.loop(0, _GROUPS)(extract)

        pltpu.sync_copy(out_v, out_hbm.at[pl.ds(base * DIM, _B_PER_W * DIM)])

    return gather_kernel


_gather = _make_gather()


def kernel(idx, context_hat):
    idx_flat = idx.reshape(BATCH).astype(jnp.int32)
    table_blocks = context_hat.reshape(-1, ROWS_PER_BLOCK * DIM)
    out_flat = _gather(idx_flat, table_blocks)
    return out_flat.reshape(BATCH, DIM)
